# baseline (speedup 1.0000x reference)
import functools

import jax
import jax.numpy as jnp
from jax import lax
from jax.experimental import pallas as pl
from jax.experimental.pallas import tpu as pltpu
from jax.experimental.pallas import tpu_sc as plsc

D = 64
B = 16384
VOCAB = 1000000

_NW = 32
_SPAN = 31232
_CW = 512
_NCH = _SPAN // _CW
_NCID = 63
_MYCAP = 768
_BCAP = 48
_SCAP = 1280
_KMAX = _SCAP // 128

_mesh = plsc.VectorSubcoreMesh(core_axis_name="c", subcore_axis_name="s")


@functools.partial(
    pl.kernel,
    mesh=_mesh,
    compiler_params=pltpu.CompilerParams(needs_layout_passes=False),
    out_type=jax.ShapeDtypeStruct((B,), jnp.float32),
    scratch_types=[
        pltpu.VMEM((B,), jnp.int32),
        pltpu.VMEM((_MYCAP,), jnp.int32),
        pltpu.VMEM((_MYCAP,), jnp.int32),
        pltpu.VMEM((_NCID, 64), jnp.int32),
        pltpu.VMEM((_NCID, 64), jnp.int32),
        pltpu.SMEM((_NCID,), jnp.int32),
        pltpu.VMEM((D, _CW), jnp.float32),
        pltpu.VMEM((D, _CW), jnp.float32),
        pltpu.VMEM((D, 128), jnp.float32),
        pltpu.VMEM((_SCAP,), jnp.float32),
        pltpu.VMEM((_SCAP,), jnp.int32),
        pltpu.VMEM((_KMAX, 128), jnp.int32),
        pltpu.VMEM((D,), jnp.float32),
        pltpu.SemaphoreType.DMA,
        pltpu.SemaphoreType.DMA,
        pltpu.SemaphoreType.DMA,
        pltpu.SemaphoreType.DMA,
    ],
)
def _mf_kernel(items_hbm, u_hbm, tt_hbm, out_hbm, idx_all, myv, myp,
               bkt_v, bkt_p, cnt_s, buf_a, buf_b, buf_t, sres, spos, spos2,
               u_v, sem_a, sem_b, sem_t, sem_s):
    wid = lax.axis_index("s") * 2 + lax.axis_index("c")
    lo = wid * _SPAN
    hi = jnp.where(wid == _NW - 1, VOCAB, lo + _SPAN)
    pltpu.sync_copy(items_hbm, idx_all)
    pltpu.sync_copy(u_hbm, u_v)
    lanes = lax.iota(jnp.int32, 16)
    zeros16 = jnp.zeros((16,), jnp.int32)
    u_vecs = [u_v[pl.ds(q * 16, 16)] for q in range(D // 16)]

    def sel_body(g, cnt):
        v16 = idx_all[pl.ds(g * 16, 16)]
        m = (v16 >= lo) & (v16 < hi)
        mi = m.astype(jnp.int32)
        cum = plsc.cumsum(mi) - mi
        pos = jnp.minimum(cnt, _MYCAP - 16) + cum
        plsc.store_scatter(myv, [pos], v16, mask=m)
        plsc.store_scatter(myp, [pos], g * 16 + lanes, mask=m)
        nm = plsc.all_reduce_population_count(m)[0]
        return jnp.minimum(cnt + nm, _MYCAP - 16)

    count = lax.fori_loop(0, B // 16, sel_body, jnp.int32(0))
    v0 = myv[pl.ds(0, 16)].at[zeros16].get(mode="promise_in_bounds")
    p0 = myp[pl.ds(0, 16)].at[zeros16].get(mode="promise_in_bounds")
    padm = lanes >= (count & 15)
    padbase = count & ~jnp.int32(15)
    plsc.store_scatter(myv, [padbase + lanes], v0, mask=padm)
    plsc.store_scatter(myp, [padbase + lanes], p0, mask=padm)
    n_my = (count + 15) >> 4

    def bkt_body(c, carry):
        def inner(i, cc):
            v16 = myv[pl.ds(i * 16, 16)]
            m = ((v16 - lo) >> 9) == c
            mi = m.astype(jnp.int32)
            cum = plsc.cumsum(mi) - mi
            bpos = jnp.minimum(cc, _BCAP - 16) + cum
            plsc.store_scatter(
                bkt_v, [jnp.full((16,), c, jnp.int32), bpos],
                v16 - (lo + c * _CW), mask=m)
            plsc.store_scatter(
                bkt_p, [jnp.full((16,), c, jnp.int32), bpos],
                myp[pl.ds(i * 16, 16)], mask=m)
            nm = plsc.all_reduce_population_count(m)[0]
            return jnp.minimum(cc + nm, _BCAP - 16)

        cnt = lax.fori_loop(0, n_my, inner, jnp.int32(0))
        cnt_s[c] = cnt

        @pl.when(cnt > 0)
        def _():
            b0 = bkt_v[c, pl.ds(0, 16)].at[zeros16].get(
                mode="promise_in_bounds")
            q0 = bkt_p[c, pl.ds(0, 16)].at[zeros16].get(
                mode="promise_in_bounds")
            pm = lanes >= (cnt & 15)
            pb = cnt & ~jnp.int32(15)
            cvec = jnp.full((16,), c, jnp.int32)
            plsc.store_scatter(bkt_v, [cvec, pb + lanes], b0, mask=pm)
            plsc.store_scatter(bkt_p, [cvec, pb + lanes], q0, mask=pm)

        return carry

    lax.fori_loop(0, _NCID, bkt_body, jnp.int32(0))

    def issue(cstart, w, buf, sem):
        start = pl.multiple_of(cstart, 128)
        pltpu.async_copy(tt_hbm.at[:, pl.ds(start, w)], buf, sem)

    def drain(w, buf, sem):
        pltpu.make_async_copy(tt_hbm.at[:, pl.ds(0, w)], buf, sem).wait()

    def process(c, buf, soff):
        ngrp = (cnt_s[c] + 15) >> 4

        def grp_body(g, off):
            offs = bkt_v[c, pl.ds(g * 16, 16)]
            acc = jnp.zeros((16,), jnp.float32)
            for f in range(D):
                vals = plsc.load_gather(
                    buf, [jnp.full((16,), f, jnp.int32), offs])
                acc = acc + vals * u_vecs[f // 16][f % 16]
            sres[pl.ds(off, 16)] = acc
            spos[pl.ds(off, 16)] = bkt_p[c, pl.ds(g * 16, 16)]
            return off + 16

        return lax.fori_loop(0, ngrp, grp_body, soff)

    issue(lo, _CW, buf_a, sem_a)

    def sweep_body(t, soff):
        c0 = 2 * t
        issue(lo + (c0 + 1) * _CW, _CW, buf_b, sem_b)
        drain(_CW, buf_a, sem_a)
        soff = process(c0, buf_a, soff)
        issue(lo + (c0 + 2) * _CW, _CW, buf_a, sem_a)
        drain(_CW, buf_b, sem_b)
        return process(c0 + 1, buf_b, soff)

    soff1 = lax.fori_loop(0, _NCH // 2, sweep_body, jnp.int32(0))

    @pl.when(wid == _NW - 1)
    def _():
        issue(lo + 61 * _CW, _CW, buf_b, sem_b)
        issue(lo + 62 * _CW, 128, buf_t, sem_t)

    drain(_CW, buf_a, sem_a)
    soff2 = process(_NCH - 1, buf_a, soff1)

    def tail_case():
        drain(_CW, buf_b, sem_b)
        s = process(61, buf_b, soff2)
        drain(128, buf_t, sem_t)
        return process(62, buf_t, s)

    staged = lax.cond(wid == _NW - 1, tail_case, lambda: soff2)

    r0 = sres[pl.ds(0, 16)].at[zeros16].get(mode="promise_in_bounds")
    q0 = spos[pl.ds(0, 16)].at[zeros16].get(mode="promise_in_bounds")
    nk = (staged + 127) >> 7

    def pad_body(t, carry):
        sres[pl.ds(t * 16, 16)] = r0
        spos[pl.ds(t * 16, 16)] = q0
        return carry

    lax.fori_loop(staged >> 4, nk * 8, pad_body, jnp.int32(0))

    def row_body(t, carry):
        spos2[t >> 3, pl.ds((t & 7) * 16, 16)] = spos[pl.ds(t * 16, 16)]
        return carry

    lax.fori_loop(0, nk * 8, row_body, jnp.int32(0))

    def scat_body(k, carry):
        pltpu.async_copy(
            sres.at[pl.ds(k * 128, 128)], out_hbm.at[spos2.at[k]], sem_s)
        return carry

    lax.fori_loop(0, nk, scat_body, jnp.int32(0))

    def wait_body(k, carry):
        pltpu.make_async_copy(
            sres.at[pl.ds(0, 128)], out_hbm.at[spos2.at[0]], sem_s).wait()
        return carry

    lax.fori_loop(0, nk, wait_body, jnp.int32(0))


def kernel(users, items, user_factors, item_factors):
    del users
    u = user_factors.reshape((D,))
    return _mf_kernel(items, u, item_factors.T)


# speedup vs baseline: 1.0198x; 1.0198x over previous
import functools

import jax
import jax.numpy as jnp
from jax import lax
from jax.experimental import pallas as pl
from jax.experimental.pallas import tpu as pltpu
from jax.experimental.pallas import tpu_sc as plsc

D = 64
B = 16384
VOCAB = 1000000

_NW = 32
_SPAN = 31232
_CW = 512
_NCH = _SPAN // _CW
_NCID = 63
_MYCAP = 768
_BCAP = 48
_SCAP = 1280
_KMAX = _SCAP // 128

_mesh = plsc.VectorSubcoreMesh(core_axis_name="c", subcore_axis_name="s")


@functools.partial(
    pl.kernel,
    mesh=_mesh,
    compiler_params=pltpu.CompilerParams(needs_layout_passes=False),
    out_type=jax.ShapeDtypeStruct((B,), jnp.float32),
    scratch_types=[
        pltpu.VMEM((B,), jnp.int32),
        pltpu.VMEM((_MYCAP,), jnp.int32),
        pltpu.VMEM((_MYCAP,), jnp.int32),
        pltpu.VMEM((_NCID, 64), jnp.int32),
        pltpu.VMEM((_NCID, 64), jnp.int32),
        pltpu.SMEM((_NCID,), jnp.int32),
        pltpu.VMEM((D, _CW), jnp.float32),
        pltpu.VMEM((D, _CW), jnp.float32),
        pltpu.VMEM((D, 128), jnp.float32),
        pltpu.VMEM((_SCAP,), jnp.float32),
        pltpu.VMEM((_SCAP,), jnp.int32),
        pltpu.VMEM((_KMAX, 128), jnp.int32),
        pltpu.VMEM((D,), jnp.float32),
        pltpu.SemaphoreType.DMA,
        pltpu.SemaphoreType.DMA,
        pltpu.SemaphoreType.DMA,
        pltpu.SemaphoreType.DMA,
    ],
)
def _mf_kernel(items_hbm, u_hbm, tt_hbm, out_hbm, idx_all, myv, myp,
               bkt_v, bkt_p, cnt_s, buf_a, buf_b, buf_t, sres, spos, spos2,
               u_v, sem_a, sem_b, sem_t, sem_s):
    wid = lax.axis_index("s") * 2 + lax.axis_index("c")
    lo = wid * _SPAN
    hi = jnp.where(wid == _NW - 1, VOCAB, lo + _SPAN)
    pltpu.sync_copy(items_hbm, idx_all)
    pltpu.sync_copy(u_hbm, u_v)
    lanes = lax.iota(jnp.int32, 16)
    zeros16 = jnp.zeros((16,), jnp.int32)
    u_vecs = [u_v[pl.ds(q * 16, 16)] for q in range(D // 16)]

    def sel_body(g, cnt):
        v16 = idx_all[pl.ds(g * 16, 16)]
        m = (v16 >= lo) & (v16 < hi)
        mi = m.astype(jnp.int32)
        cum = plsc.cumsum(mi) - mi
        pos = jnp.minimum(cnt, _MYCAP - 16) + cum
        plsc.store_scatter(myv, [pos], v16, mask=m)
        plsc.store_scatter(myp, [pos], g * 16 + lanes, mask=m)
        nm = plsc.all_reduce_population_count(m)[0]
        return jnp.minimum(cnt + nm, _MYCAP - 16)

    count = lax.fori_loop(0, B // 16, sel_body, jnp.int32(0))
    v0 = myv[pl.ds(0, 16)].at[zeros16].get(mode="promise_in_bounds")
    p0 = myp[pl.ds(0, 16)].at[zeros16].get(mode="promise_in_bounds")
    padm = lanes >= (count & 15)
    padbase = count & ~jnp.int32(15)
    plsc.store_scatter(myv, [padbase + lanes], v0, mask=padm)
    plsc.store_scatter(myp, [padbase + lanes], p0, mask=padm)
    n_my = (count + 15) >> 4

    def bkt_body(c, carry):
        def inner(i, cc):
            v16 = myv[pl.ds(i * 16, 16)]
            m = ((v16 - lo) >> 9) == c
            mi = m.astype(jnp.int32)
            cum = plsc.cumsum(mi) - mi
            bpos = jnp.minimum(cc, _BCAP - 16) + cum
            plsc.store_scatter(
                bkt_v, [jnp.full((16,), c, jnp.int32), bpos],
                v16 - (lo + c * _CW), mask=m)
            plsc.store_scatter(
                bkt_p, [jnp.full((16,), c, jnp.int32), bpos],
                myp[pl.ds(i * 16, 16)], mask=m)
            nm = plsc.all_reduce_population_count(m)[0]
            return jnp.minimum(cc + nm, _BCAP - 16)

        cnt = lax.fori_loop(0, n_my, inner, jnp.int32(0))
        cnt_s[c] = cnt

        @pl.when(cnt > 0)
        def _():
            b0 = bkt_v[c, pl.ds(0, 16)].at[zeros16].get(
                mode="promise_in_bounds")
            q0 = bkt_p[c, pl.ds(0, 16)].at[zeros16].get(
                mode="promise_in_bounds")
            pm = lanes >= (cnt & 15)
            pb = cnt & ~jnp.int32(15)
            cvec = jnp.full((16,), c, jnp.int32)
            plsc.store_scatter(bkt_v, [cvec, pb + lanes], b0, mask=pm)
            plsc.store_scatter(bkt_p, [cvec, pb + lanes], q0, mask=pm)

        return carry

    lax.fori_loop(0, _NCID, bkt_body, jnp.int32(0))

    def issue(cstart, w, buf, sem):
        start = pl.multiple_of(cstart, 128)
        for g in range(8):
            pltpu.async_copy(
                tt_hbm.at[pl.ds(g * 8, 8), pl.ds(start, w)],
                buf.at[pl.ds(g * 8, 8)], sem)

    def drain(w, buf, sem):
        for g in range(8):
            pltpu.make_async_copy(
                tt_hbm.at[pl.ds(0, 8), pl.ds(0, w)],
                buf.at[pl.ds(g * 8, 8)], sem).wait()

    def process(c, buf, soff):
        ngrp = (cnt_s[c] + 15) >> 4

        def grp_body(g, off):
            offs = bkt_v[c, pl.ds(g * 16, 16)]
            acc = jnp.zeros((16,), jnp.float32)
            for f in range(D):
                vals = plsc.load_gather(
                    buf, [jnp.full((16,), f, jnp.int32), offs])
                acc = acc + vals * u_vecs[f // 16][f % 16]
            sres[pl.ds(off, 16)] = acc
            spos[pl.ds(off, 16)] = bkt_p[c, pl.ds(g * 16, 16)]
            return off + 16

        return lax.fori_loop(0, ngrp, grp_body, soff)

    issue(lo, _CW, buf_a, sem_a)

    def sweep_body(t, soff):
        c0 = 2 * t
        issue(lo + (c0 + 1) * _CW, _CW, buf_b, sem_b)
        drain(_CW, buf_a, sem_a)
        soff = process(c0, buf_a, soff)
        issue(lo + (c0 + 2) * _CW, _CW, buf_a, sem_a)
        drain(_CW, buf_b, sem_b)
        return process(c0 + 1, buf_b, soff)

    soff1 = lax.fori_loop(0, _NCH // 2, sweep_body, jnp.int32(0))

    @pl.when(wid == _NW - 1)
    def _():
        issue(lo + 61 * _CW, _CW, buf_b, sem_b)
        issue(lo + 62 * _CW, 128, buf_t, sem_t)

    drain(_CW, buf_a, sem_a)
    soff2 = process(_NCH - 1, buf_a, soff1)

    def tail_case():
        drain(_CW, buf_b, sem_b)
        s = process(61, buf_b, soff2)
        drain(128, buf_t, sem_t)
        return process(62, buf_t, s)

    staged = lax.cond(wid == _NW - 1, tail_case, lambda: soff2)

    r0 = sres[pl.ds(0, 16)].at[zeros16].get(mode="promise_in_bounds")
    q0 = spos[pl.ds(0, 16)].at[zeros16].get(mode="promise_in_bounds")
    nk = (staged + 127) >> 7

    def pad_body(t, carry):
        sres[pl.ds(t * 16, 16)] = r0
        spos[pl.ds(t * 16, 16)] = q0
        return carry

    lax.fori_loop(staged >> 4, nk * 8, pad_body, jnp.int32(0))

    def row_body(t, carry):
        spos2[t >> 3, pl.ds((t & 7) * 16, 16)] = spos[pl.ds(t * 16, 16)]
        return carry

    lax.fori_loop(0, nk * 8, row_body, jnp.int32(0))

    def scat_body(k, carry):
        pltpu.async_copy(
            sres.at[pl.ds(k * 128, 128)], out_hbm.at[spos2.at[k]], sem_s)
        return carry

    lax.fori_loop(0, nk, scat_body, jnp.int32(0))

    def wait_body(k, carry):
        pltpu.make_async_copy(
            sres.at[pl.ds(0, 128)], out_hbm.at[spos2.at[0]], sem_s).wait()
        return carry

    lax.fori_loop(0, nk, wait_body, jnp.int32(0))


def kernel(users, items, user_factors, item_factors):
    del users
    u = user_factors.reshape((D,))
    return _mf_kernel(items, u, item_factors.T)


# speedup vs baseline: 1.8982x; 1.8615x over previous
import functools

import jax
import jax.numpy as jnp
from jax import lax
from jax.experimental import pallas as pl
from jax.experimental.pallas import tpu as pltpu
from jax.experimental.pallas import tpu_sc as plsc

D = 64
B = 16384

_NC = 2
_NS = 16
_NW = _NC * _NS
_BPW = B // _NW
_NBUF = 8
_NIT = _BPW // _NBUF

_mesh = plsc.VectorSubcoreMesh(core_axis_name="c", subcore_axis_name="s")


@functools.partial(
    pl.kernel,
    mesh=_mesh,
    compiler_params=pltpu.CompilerParams(needs_layout_passes=False),
    out_type=jax.ShapeDtypeStruct((B,), jnp.float32),
    scratch_types=[
        pltpu.VMEM((_BPW + 16,), jnp.int32),
        pltpu.VMEM((_NBUF, D, 128), jnp.float32),
        pltpu.VMEM((D,), jnp.float32),
        pltpu.VMEM((_BPW,), jnp.float32),
    ]
    + [pltpu.SemaphoreType.DMA] * _NBUF,
)
def _mf_kernel(items_hbm, u_hbm, tt_hbm, out_hbm, idx_v, bufs, u_v, out_v,
               *sems):
    wid = lax.axis_index("s") * _NC + lax.axis_index("c")
    base = wid * _BPW
    pltpu.sync_copy(items_hbm.at[pl.ds(base, _BPW)], idx_v.at[pl.ds(0, _BPW)])
    pltpu.sync_copy(u_hbm, u_v)
    lanes = lax.iota(jnp.int32, 16)
    u_vecs = [u_v[pl.ds(q * 16, 16)] for q in range(D // 16)]

    def issue(start_scalar, slot, sem):
        start = pl.multiple_of(start_scalar, 128)
        pltpu.async_copy(tt_hbm.at[:, pl.ds(start, 128)], bufs.at[slot], sem)

    starts0 = lax.bitwise_and(idx_v[pl.ds(0, 16)], jnp.int32(~127))
    for j in range(_NBUF):
        issue(starts0[j], j, sems[j])

    def body(t, carry):
        iv = idx_v[pl.ds(t * _NBUF, 16)]
        starts = lax.bitwise_and(iv, jnp.int32(~127))
        loffs = lax.bitwise_and(iv, jnp.int32(112))
        l15 = lax.bitwise_and(iv, jnp.int32(15))
        res = jnp.zeros((16,), jnp.float32)
        for j in range(_NBUF):
            pltpu.make_async_copy(
                tt_hbm.at[:, pl.ds(0, 128)], bufs.at[j], sems[j]
            ).wait()
            loff = loffs[j]
            acc = jnp.zeros((16,), jnp.float32)
            for f in range(D):
                acc = acc + (bufs[j, f, pl.ds(loff, 16)]
                             * u_vecs[f // 16][f % 16])

            @pl.when(t + 1 < _NIT)
            def _():
                issue(starts[j + 8], j, sems[j])

            pick = acc.at[jnp.full((16,), l15[j], jnp.int32)].get(
                mode="promise_in_bounds")
            res = jnp.where(lanes == j, pick, res)
        plsc.store_scatter(
            out_v,
            [t * _NBUF + lax.bitwise_and(lanes, jnp.int32(7))],
            res,
            mask=lanes < _NBUF,
        )
        return carry

    lax.fori_loop(0, _NIT, body, 0)
    pltpu.sync_copy(out_v, out_hbm.at[pl.ds(base, _BPW)])


def kernel(users, items, user_factors, item_factors):
    del users
    u = user_factors.reshape((D,))
    return _mf_kernel(items, u, item_factors.T)
